# baseline (speedup 1.0000x reference)
import functools

import jax
import jax.numpy as jnp
from jax import lax
from jax.experimental import pallas as pl
from jax.experimental.pallas import tpu as pltpu
from jax.experimental.pallas import tpu_sc as plsc

_LANES = 16


def _lane_total(v):
    lanes = lax.iota(jnp.int32, _LANES)
    dnums = lax.GatherDimensionNumbers(
        offset_dims=(), collapsed_slice_dims=(0,), start_index_map=(0,)
    )
    for sh in (8, 4, 2, 1):
        idx = lax.rem(lanes + sh, jnp.full((_LANES,), _LANES, jnp.int32))
        rot = lax.gather(
            v,
            idx[:, None],
            dnums,
            slice_sizes=(1,),
            mode=lax.GatherScatterMode.PROMISE_IN_BOUNDS,
        )
        v = v + rot
    return v[0]


def kernel(hidden_states, attention_mask):
    B, S, H = hidden_states.shape
    mesh = plsc.VectorSubcoreMesh(
        core_axis_name="c", subcore_axis_name="s", num_cores=1
    )

    @functools.partial(
        pl.kernel,
        mesh=mesh,
        out_type=jax.ShapeDtypeStruct((B, H), hidden_states.dtype),
        scratch_types=[
            pltpu.VMEM((S,), jnp.int32),
        ],
    )
    def _sc(hs_hbm, mask_hbm, out_hbm, mask_v):
        sid = lax.axis_index("s")

        @pl.when(sid < B)
        def _():
            b = sid
            pltpu.sync_copy(mask_hbm.at[b], mask_v)

            unroll = 8
            zero = jnp.zeros((_LANES,), jnp.int32)

            def step(i, accs):
                base = i * (_LANES * unroll)
                return tuple(
                    accs[j] + mask_v[pl.ds(base + j * _LANES, _LANES)]
                    for j in range(unroll)
                )

            accs = lax.fori_loop(
                0, S // (_LANES * unroll), step, (zero,) * unroll
            )
            acc = accs[0]
            for j in range(1, unroll):
                acc = acc + accs[j]
            idx = _lane_total(acc) - 1
            pltpu.sync_copy(hs_hbm.at[b, idx], out_hbm.at[b])

    return _sc(hidden_states, attention_mask)


# speedup vs baseline: 1.0097x; 1.0097x over previous
import functools

import jax
import jax.numpy as jnp
from jax import lax
from jax.experimental import pallas as pl
from jax.experimental.pallas import tpu as pltpu
from jax.experimental.pallas import tpu_sc as plsc

_LANES = 16


def _lane_total(v):
    lanes = lax.iota(jnp.int32, _LANES)
    dnums = lax.GatherDimensionNumbers(
        offset_dims=(), collapsed_slice_dims=(0,), start_index_map=(0,)
    )
    for sh in (8, 4, 2, 1):
        idx = lax.rem(lanes + sh, jnp.full((_LANES,), _LANES, jnp.int32))
        rot = lax.gather(
            v,
            idx[:, None],
            dnums,
            slice_sizes=(1,),
            mode=lax.GatherScatterMode.PROMISE_IN_BOUNDS,
        )
        v = v + rot
    return v[0]


def kernel(hidden_states, attention_mask):
    B, S, H = hidden_states.shape
    NSUB = 16
    PER = NSUB // B
    SQ = S // PER
    mesh = plsc.VectorSubcoreMesh(
        core_axis_name="c", subcore_axis_name="s", num_cores=1
    )

    @functools.partial(
        pl.kernel,
        mesh=mesh,
        out_type=jax.ShapeDtypeStruct((B, H), hidden_states.dtype),
        scratch_types=[
            pltpu.VMEM((SQ,), jnp.int32),
            pltpu.VMEM((_LANES,), jnp.int32),
            pltpu.VMEM((PER, _LANES), jnp.int32),
            pltpu.VMEM_SHARED((NSUB, _LANES), jnp.int32),
        ],
    )
    def _sc(hs_hbm, mask_hbm, out_hbm, mask_v, part_v, quad_v, shared):
        sid = lax.axis_index("s")
        b = sid // PER
        q = lax.rem(sid, PER)

        pltpu.sync_copy(mask_hbm.at[b, pl.ds(q * SQ, SQ)], mask_v)

        unroll = 8
        zero = jnp.zeros((_LANES,), jnp.int32)
        accs = [zero] * unroll
        for i in range(SQ // _LANES):
            accs[i % unroll] = accs[i % unroll] + mask_v[
                pl.ds(i * _LANES, _LANES)
            ]
        acc = accs[0]
        for j in range(1, unroll):
            acc = acc + accs[j]
        part_v[...] = acc
        pltpu.sync_copy(part_v, shared.at[sid])
        plsc.subcore_barrier()

        @pl.when(q == 0)
        def _():
            pltpu.sync_copy(shared.at[pl.ds(b * PER, PER)], quad_v)
            total = quad_v[0]
            for j in range(1, PER):
                total = total + quad_v[j]
            idx = _lane_total(total) - 1
            pltpu.sync_copy(hs_hbm.at[b, idx], out_hbm.at[b])

    return _sc(hidden_states, attention_mask)
